# baseline (speedup 1.0000x reference)
import functools

import numpy as np
import jax
import jax.numpy as jnp
from jax.experimental import pallas as pl

_T_PATCH = (1, 1, 2, 2, 2, 4, 4, 4, 4, 4)
_V_PATCH = (1, 2, 3, 4, 5, 6, 8, 10, 13, 16)
_N_QRESI = 4
_NC = 8192
_C = 256
_KC = 1024
_HP = jax.lax.Precision.HIGHEST


def _area_matrix(n_in, n_out):
    M = np.zeros((n_out, n_in), dtype=np.float32)
    for i in range(n_out):
        s = (i * n_in) // n_out
        e = int(np.ceil((i + 1) * n_in / n_out))
        M[i, s:e] = 1.0 / (e - s)
    return jnp.asarray(M)


def _linear_matrix(n_in, n_out):
    M = np.zeros((n_out, n_in), dtype=np.float32)
    if n_in == 1:
        M[:, 0] = 1.0
        return jnp.asarray(M)
    scale = n_in / n_out
    for i in range(n_out):
        src = max((i + 0.5) * scale - 0.5, 0.0)
        i0 = min(int(np.floor(src)), n_in - 1)
        i1 = min(i0 + 1, n_in - 1)
        lam = src - i0
        M[i, i0] += 1.0 - lam
        M[i, i1] += lam
    return jnp.asarray(M)


def _resize3d(x, size, mat_fn):
    t, h, w = size
    x = jnp.einsum('bcthw,ut->bcuhw', x.astype(jnp.bfloat16),
                   mat_fn(x.shape[2], t).astype(jnp.bfloat16),
                   preferred_element_type=jnp.float32)
    x = jnp.einsum('bcthw,uh->bctuw', x.astype(jnp.bfloat16),
                   mat_fn(x.shape[3], h).astype(jnp.bfloat16),
                   preferred_element_type=jnp.float32)
    x = jnp.einsum('bcthw,uw->bcthu', x.astype(jnp.bfloat16),
                   mat_fn(x.shape[4], w).astype(jnp.bfloat16),
                   preferred_element_type=jnp.float32)
    return x


def _conv3d(x, w, b):
    y = jax.lax.conv_general_dilated(x, w, (1, 1, 1), 'SAME',
                                     dimension_numbers=('NCDHW', 'OIDHW', 'NCDHW'))
    return y + b[None, :, None, None, None]


def _vq_body(N, rd_ref, zn_ref, e_ref, et16_ref, en_ref, h_ref):
    rd16 = rd_ref[...].astype(jnp.bfloat16)
    zn = zn_ref[...]
    best_m = None
    best_a = None
    for c in range(_NC // _KC):
        s = jnp.dot(rd16, et16_ref[:, c * _KC:(c + 1) * _KC],
                    preferred_element_type=jnp.float32)
        d = (zn + en_ref[:, c * _KC:(c + 1) * _KC]) - 2.0 * s
        m = jnp.min(d, axis=1, keepdims=True)
        io = jax.lax.broadcasted_iota(jnp.int32, (N, _KC), 1)
        a = jnp.min(jnp.where(d == m, io, _KC), axis=1, keepdims=True) + c * _KC
        if best_m is None:
            best_m, best_a = m, a
        else:
            take = m < best_m
            best_m = jnp.where(take, m, best_m)
            best_a = jnp.where(take, a, best_a)
    cand = None
    for c in range(_NC // _KC):
        io = jax.lax.broadcasted_iota(jnp.int32, (N, _KC), 1) + c * _KC
        oh = (best_a == io).astype(jnp.float32)
        part = jnp.dot(oh, e_ref[c * _KC:(c + 1) * _KC, :], precision=_HP,
                       preferred_element_type=jnp.float32)
        cand = part if cand is None else cand + part
    h_ref[...] = cand


def _vq_lookup(rd_flat, zn, e, et16, en):
    N = rd_flat.shape[0]
    return pl.pallas_call(
        functools.partial(_vq_body, N),
        out_shape=jax.ShapeDtypeStruct((N, _C), jnp.float32),
    )(rd_flat, zn, e, et16, en)


def kernel(z, embeddings, qresi_w, qresi_b):
    B, C, T, H, W = z.shape
    et16 = embeddings.T.astype(jnp.bfloat16)
    en = jnp.sum(embeddings * embeddings, axis=1)[None, :]
    accu = jnp.zeros_like(z)
    scale_num = len(_V_PATCH)
    ticks = np.linspace(1.0 / 3.0 / _N_QRESI, 1.0 - 1.0 / 3.0 / _N_QRESI, _N_QRESI)
    commitment = jnp.float32(0.0)
    for si, (tpn, pn) in enumerate(zip(_T_PATCH, _V_PATCH)):
        tpn = min(tpn, T)
        rest = z - accu
        if si != scale_num - 1:
            rest = _resize3d(rest, (tpn, pn, pn), _area_matrix)
        z_NC = jnp.transpose(rest, (0, 2, 3, 4, 1)).reshape(-1, C)
        zn = jnp.sum(z_NC * z_NC, axis=1, keepdims=True)
        hc = _vq_lookup(z_NC, zn, embeddings, et16, en)
        h = hc.reshape(rest.shape[0], rest.shape[2], rest.shape[3], rest.shape[4], C)
        h = jnp.transpose(h, (0, 4, 1, 2, 3))
        h = _resize3d(h, (T, H, W), _linear_matrix)
        qi = int(np.argmin(np.abs(ticks - si / max(1, scale_num - 1))))
        h = h * 0.5 + _conv3d(h, qresi_w[qi], qresi_b[qi]) * 0.5
        accu = accu + h
        commitment = commitment + 0.25 * jnp.mean((accu - z) ** 2)
    return accu, commitment


# speedup vs baseline: 1.1662x; 1.1662x over previous
import functools

import numpy as np
import jax
import jax.numpy as jnp
from jax.experimental import pallas as pl

_T_PATCH = (1, 1, 2, 2, 2, 4, 4, 4, 4, 4)
_V_PATCH = (1, 2, 3, 4, 5, 6, 8, 10, 13, 16)
_N_QRESI = 4
_NC = 8192
_C = 256
_KC = 1024
_HP = jax.lax.Precision.HIGHEST


def _area_matrix(n_in, n_out):
    M = np.zeros((n_out, n_in), dtype=np.float32)
    for i in range(n_out):
        s = (i * n_in) // n_out
        e = int(np.ceil((i + 1) * n_in / n_out))
        M[i, s:e] = 1.0 / (e - s)
    return jnp.asarray(M)


def _linear_matrix(n_in, n_out):
    M = np.zeros((n_out, n_in), dtype=np.float32)
    if n_in == 1:
        M[:, 0] = 1.0
        return jnp.asarray(M)
    scale = n_in / n_out
    for i in range(n_out):
        src = max((i + 0.5) * scale - 0.5, 0.0)
        i0 = min(int(np.floor(src)), n_in - 1)
        i1 = min(i0 + 1, n_in - 1)
        lam = src - i0
        M[i, i0] += 1.0 - lam
        M[i, i1] += lam
    return jnp.asarray(M)


def _resize3d(x, size, mat_fn):
    t, h, w = size
    x = jnp.einsum('bcthw,ut->bcuhw', x.astype(jnp.bfloat16),
                   mat_fn(x.shape[2], t).astype(jnp.bfloat16),
                   preferred_element_type=jnp.float32)
    x = jnp.einsum('bcthw,uh->bctuw', x.astype(jnp.bfloat16),
                   mat_fn(x.shape[3], h).astype(jnp.bfloat16),
                   preferred_element_type=jnp.float32)
    x = jnp.einsum('bcthw,uw->bcthu', x.astype(jnp.bfloat16),
                   mat_fn(x.shape[4], w).astype(jnp.bfloat16),
                   preferred_element_type=jnp.float32)
    return x


def _conv3d(x, w, b):
    y = jax.lax.conv_general_dilated(x, w, (1, 1, 1), 'SAME',
                                     dimension_numbers=('NCDHW', 'OIDHW', 'NCDHW'))
    return y + b[None, :, None, None, None]


def _vq_body(N, rd_ref, zn_ref, et16_ref, en_ref, idx_ref):
    rd16 = rd_ref[...].astype(jnp.bfloat16)
    zn = zn_ref[...]
    best_m = None
    best_a = None
    for c in range(_NC // _KC):
        s = jnp.dot(rd16, et16_ref[:, c * _KC:(c + 1) * _KC],
                    preferred_element_type=jnp.float32)
        d = (zn + en_ref[:, c * _KC:(c + 1) * _KC]) - 2.0 * s
        m = jnp.min(d, axis=1, keepdims=True)
        io = jax.lax.broadcasted_iota(jnp.int32, (N, _KC), 1)
        a = jnp.min(jnp.where(d == m, io, _KC), axis=1, keepdims=True) + c * _KC
        if best_m is None:
            best_m, best_a = m, a
        else:
            take = m < best_m
            best_m = jnp.where(take, m, best_m)
            best_a = jnp.where(take, a, best_a)
    idx_ref[...] = best_a


def _vq_search(rd_flat, zn, et16, en):
    N = rd_flat.shape[0]
    return pl.pallas_call(
        functools.partial(_vq_body, N),
        out_shape=jax.ShapeDtypeStruct((N, 1), jnp.int32),
    )(rd_flat, zn, et16, en)


def kernel(z, embeddings, qresi_w, qresi_b):
    B, C, T, H, W = z.shape
    et16 = embeddings.T.astype(jnp.bfloat16)
    en = jnp.sum(embeddings * embeddings, axis=1)[None, :]
    accu = jnp.zeros_like(z)
    scale_num = len(_V_PATCH)
    ticks = np.linspace(1.0 / 3.0 / _N_QRESI, 1.0 - 1.0 / 3.0 / _N_QRESI, _N_QRESI)
    commitment = jnp.float32(0.0)
    for si, (tpn, pn) in enumerate(zip(_T_PATCH, _V_PATCH)):
        tpn = min(tpn, T)
        rest = z - accu
        if si != scale_num - 1:
            rest = _resize3d(rest, (tpn, pn, pn), _area_matrix)
        z_NC = jnp.transpose(rest, (0, 2, 3, 4, 1)).reshape(-1, C)
        zn = jnp.sum(z_NC * z_NC, axis=1, keepdims=True)
        idx = _vq_search(z_NC, zn, et16, en)
        hc = jnp.take(embeddings, idx[:, 0], axis=0)
        h = hc.reshape(rest.shape[0], rest.shape[2], rest.shape[3], rest.shape[4], C)
        h = jnp.transpose(h, (0, 4, 1, 2, 3))
        h = _resize3d(h, (T, H, W), _linear_matrix)
        qi = int(np.argmin(np.abs(ticks - si / max(1, scale_num - 1))))
        h = h * 0.5 + _conv3d(h, qresi_w[qi], qresi_b[qi]) * 0.5
        accu = accu + h
        commitment = commitment + 0.25 * jnp.mean((accu - z) ** 2)
    return accu, commitment


# speedup vs baseline: 1.1754x; 1.0079x over previous
import functools

import numpy as np
import jax
import jax.numpy as jnp
from jax.experimental import pallas as pl

_T_PATCH = (1, 1, 2, 2, 2, 4, 4, 4, 4, 4)
_V_PATCH = (1, 2, 3, 4, 5, 6, 8, 10, 13, 16)
_N_QRESI = 4
_NC = 8192
_C = 256
_KC = 2048
_HP = jax.lax.Precision.HIGHEST


def _area_matrix(n_in, n_out):
    M = np.zeros((n_out, n_in), dtype=np.float32)
    for i in range(n_out):
        s = (i * n_in) // n_out
        e = int(np.ceil((i + 1) * n_in / n_out))
        M[i, s:e] = 1.0 / (e - s)
    return jnp.asarray(M)


def _linear_matrix(n_in, n_out):
    M = np.zeros((n_out, n_in), dtype=np.float32)
    if n_in == 1:
        M[:, 0] = 1.0
        return jnp.asarray(M)
    scale = n_in / n_out
    for i in range(n_out):
        src = max((i + 0.5) * scale - 0.5, 0.0)
        i0 = min(int(np.floor(src)), n_in - 1)
        i1 = min(i0 + 1, n_in - 1)
        lam = src - i0
        M[i, i0] += 1.0 - lam
        M[i, i1] += lam
    return jnp.asarray(M)


def _resize3d(x, size, mat_fn):
    t, h, w = size
    x = jnp.einsum('bcthw,ut->bcuhw', x.astype(jnp.bfloat16),
                   mat_fn(x.shape[2], t).astype(jnp.bfloat16),
                   preferred_element_type=jnp.float32)
    x = jnp.einsum('bcthw,uh->bctuw', x.astype(jnp.bfloat16),
                   mat_fn(x.shape[3], h).astype(jnp.bfloat16),
                   preferred_element_type=jnp.float32)
    x = jnp.einsum('bcthw,uw->bcthu', x.astype(jnp.bfloat16),
                   mat_fn(x.shape[4], w).astype(jnp.bfloat16),
                   preferred_element_type=jnp.float32)
    return x


def _conv3d(x, w, b):
    y = jax.lax.conv_general_dilated(x, w, (1, 1, 1), 'SAME',
                                     dimension_numbers=('NCDHW', 'OIDHW', 'NCDHW'))
    return y + b[None, :, None, None, None]


def _vq_body(N, rd_ref, zn_ref, et16_ref, en_ref, idx_ref):
    rd16 = rd_ref[...].astype(jnp.bfloat16)
    zn = zn_ref[...]
    best_m = None
    best_a = None
    for c in range(_NC // _KC):
        s = jnp.dot(rd16, et16_ref[:, c * _KC:(c + 1) * _KC],
                    preferred_element_type=jnp.float32)
        d = (zn + en_ref[:, c * _KC:(c + 1) * _KC]) - 2.0 * s
        m = jnp.min(d, axis=1, keepdims=True)
        io = jax.lax.broadcasted_iota(jnp.int32, (N, _KC), 1)
        a = jnp.min(jnp.where(d == m, io, _KC), axis=1, keepdims=True) + c * _KC
        if best_m is None:
            best_m, best_a = m, a
        else:
            take = m < best_m
            best_m = jnp.where(take, m, best_m)
            best_a = jnp.where(take, a, best_a)
    idx_ref[...] = best_a


def _vq_search(rd_flat, zn, et16, en):
    N = rd_flat.shape[0]
    return pl.pallas_call(
        functools.partial(_vq_body, N),
        out_shape=jax.ShapeDtypeStruct((N, 1), jnp.int32),
    )(rd_flat, zn, et16, en)


def kernel(z, embeddings, qresi_w, qresi_b):
    B, C, T, H, W = z.shape
    et16 = embeddings.T.astype(jnp.bfloat16)
    en = jnp.sum(embeddings * embeddings, axis=1)[None, :]
    accu = jnp.zeros_like(z)
    scale_num = len(_V_PATCH)
    ticks = np.linspace(1.0 / 3.0 / _N_QRESI, 1.0 - 1.0 / 3.0 / _N_QRESI, _N_QRESI)
    commitment = jnp.float32(0.0)
    for si, (tpn, pn) in enumerate(zip(_T_PATCH, _V_PATCH)):
        tpn = min(tpn, T)
        rest = z - accu
        if si != scale_num - 1:
            rest = _resize3d(rest, (tpn, pn, pn), _area_matrix)
        z_NC = jnp.transpose(rest, (0, 2, 3, 4, 1)).reshape(-1, C)
        zn = jnp.sum(z_NC * z_NC, axis=1, keepdims=True)
        idx = _vq_search(z_NC, zn, et16, en)
        hc = jnp.take(embeddings, idx[:, 0], axis=0)
        h = hc.reshape(rest.shape[0], rest.shape[2], rest.shape[3], rest.shape[4], C)
        h = jnp.transpose(h, (0, 4, 1, 2, 3))
        h = _resize3d(h, (T, H, W), _linear_matrix)
        qi = int(np.argmin(np.abs(ticks - si / max(1, scale_num - 1))))
        h = h * 0.5 + _conv3d(h, qresi_w[qi], qresi_b[qi]) * 0.5
        accu = accu + h
        commitment = commitment + 0.25 * jnp.mean((accu - z) ** 2)
    return accu, commitment


# speedup vs baseline: 1.1766x; 1.0010x over previous
import functools

import numpy as np
import jax
import jax.numpy as jnp
from jax.experimental import pallas as pl

_T_PATCH = (1, 1, 2, 2, 2, 4, 4, 4, 4, 4)
_V_PATCH = (1, 2, 3, 4, 5, 6, 8, 10, 13, 16)
_N_QRESI = 4
_NC = 8192
_C = 256
_KC = 2048


def _area_matrix(n_in, n_out):
    M = np.zeros((n_out, n_in), dtype=np.float32)
    for i in range(n_out):
        s = (i * n_in) // n_out
        e = int(np.ceil((i + 1) * n_in / n_out))
        M[i, s:e] = 1.0 / (e - s)
    return jnp.asarray(M)


def _linear_matrix(n_in, n_out):
    M = np.zeros((n_out, n_in), dtype=np.float32)
    if n_in == 1:
        M[:, 0] = 1.0
        return jnp.asarray(M)
    scale = n_in / n_out
    for i in range(n_out):
        src = max((i + 0.5) * scale - 0.5, 0.0)
        i0 = min(int(np.floor(src)), n_in - 1)
        i1 = min(i0 + 1, n_in - 1)
        lam = src - i0
        M[i, i0] += 1.0 - lam
        M[i, i1] += lam
    return jnp.asarray(M)


def _resize3d(x, size, mat_fn):
    t, h, w = size
    x = jnp.einsum('bcthw,ut->bcuhw', x.astype(jnp.bfloat16),
                   mat_fn(x.shape[2], t).astype(jnp.bfloat16),
                   preferred_element_type=jnp.float32)
    x = jnp.einsum('bcthw,uh->bctuw', x.astype(jnp.bfloat16),
                   mat_fn(x.shape[3], h).astype(jnp.bfloat16),
                   preferred_element_type=jnp.float32)
    x = jnp.einsum('bcthw,uw->bcthu', x.astype(jnp.bfloat16),
                   mat_fn(x.shape[4], w).astype(jnp.bfloat16),
                   preferred_element_type=jnp.float32)
    return x


def _conv3d(x, w, b):
    y = jax.lax.conv_general_dilated(x, w, (1, 1, 1), 'SAME',
                                     dimension_numbers=('NCDHW', 'OIDHW', 'NCDHW'))
    return y + b[None, :, None, None, None]


def _vq_body(N, rd_ref, zn_ref, et16_ref, en_ref, idx_ref):
    rd16 = rd_ref[...].astype(jnp.bfloat16)
    zn = zn_ref[...]
    best_m = None
    best_a = None
    for c in range(_NC // _KC):
        s = jnp.dot(rd16, et16_ref[:, c * _KC:(c + 1) * _KC],
                    preferred_element_type=jnp.float32)
        d = (zn + en_ref[:, c * _KC:(c + 1) * _KC]) - 2.0 * s
        m = jnp.min(d, axis=1, keepdims=True)
        io = jax.lax.broadcasted_iota(jnp.int32, (N, _KC), 1)
        a = jnp.min(jnp.where(d == m, io, _KC), axis=1, keepdims=True) + c * _KC
        if best_m is None:
            best_m, best_a = m, a
        else:
            take = m < best_m
            best_m = jnp.where(take, m, best_m)
            best_a = jnp.where(take, a, best_a)
    idx_ref[...] = best_a


def _vq_search(rd_flat, zn, et16, en):
    N = rd_flat.shape[0]
    return pl.pallas_call(
        functools.partial(_vq_body, N),
        out_shape=jax.ShapeDtypeStruct((N, 1), jnp.int32),
    )(rd_flat, zn, et16, en)


def kernel(z, embeddings, qresi_w, qresi_b):
    B, C, T, H, W = z.shape
    et16 = embeddings.T.astype(jnp.bfloat16)
    en = jnp.sum(embeddings * embeddings, axis=1)[None, :]
    accu = jnp.zeros_like(z)
    scale_num = len(_V_PATCH)
    ticks = np.linspace(1.0 / 3.0 / _N_QRESI, 1.0 - 1.0 / 3.0 / _N_QRESI, _N_QRESI)
    commitment = jnp.float32(0.0)
    for si, (tpn, pn) in enumerate(zip(_T_PATCH, _V_PATCH)):
        tpn = min(tpn, T)
        rest = z - accu
        if si != scale_num - 1:
            rest = _resize3d(rest, (tpn, pn, pn), _area_matrix)
        z_NC = jnp.transpose(rest, (0, 2, 3, 4, 1)).reshape(-1, C)
        zn = jnp.sum(z_NC * z_NC, axis=1, keepdims=True)
        idx = _vq_search(z_NC, zn, et16, en)
        hc = jnp.take(embeddings, idx[:, 0], axis=0)
        h = hc.reshape(rest.shape[0], rest.shape[2], rest.shape[3], rest.shape[4], C)
        h = jnp.transpose(h, (0, 4, 1, 2, 3))
        h = _resize3d(h, (T, H, W), _linear_matrix)
        qi = int(np.argmin(np.abs(ticks - si / max(1, scale_num - 1))))
        h = h * 0.5 + _conv3d(h, qresi_w[qi], qresi_b[qi]) * 0.5
        accu = accu + h
        commitment = commitment + 0.25 * jnp.mean((accu - z) ** 2)
    return accu, commitment
